# Initial kernel scaffold; baseline (speedup 1.0000x reference)
import jax
import jax.numpy as jnp
from jax.experimental import pallas as pl


def kernel(x, edge_index, W1, b1, W2, b2):
    raise NotImplementedError("write your pallas kernel here")


# speedup vs baseline: 13.9157x; 13.9157x over previous
import functools

import jax
import jax.numpy as jnp
from jax import lax
from jax.experimental import pallas as pl
from jax.experimental.pallas import tpu as pltpu
from jax.experimental.pallas import tpu_sc as plsc

_N = 10000
_NPAD = 10240
_K = 128
_NW = 32
_NSUB = 16
_BLK = 1000


def _sc_degree(dstp):
    ch = dstp.shape[1]
    rpt = _NPAD // _NSUB
    mesh = plsc.VectorSubcoreMesh(core_axis_name="c", subcore_axis_name="s")

    @functools.partial(
        pl.kernel,
        out_type=jax.ShapeDtypeStruct((2, _NPAD, 16), jnp.float32),
        mesh=mesh,
        scratch_types=[
            pltpu.VMEM((ch, _K), jnp.int32),
            pltpu.VMEM((_K, 16), jnp.float32),
            pltpu.VMEM((16, 16), jnp.float32),
            pltpu.VMEM_SHARED((_NPAD, 16), jnp.float32),
        ],
    )
    def k(dst_h, out_h, dst_v, ones_v, zero_v, acc):
        cid = lax.axis_index("c")
        sid = lax.axis_index("s")
        wid = cid * _NSUB + sid

        one = jnp.ones((16,), jnp.float32)
        zero = jnp.zeros((16,), jnp.float32)
        for r in range(_K):
            ones_v[r, pl.ds(0, 16)] = one
        for r in range(16):
            zero_v[r, pl.ds(0, 16)] = zero

        pltpu.sync_copy(dst_h.at[wid], dst_v)

        def zbody(z, c):
            pltpu.sync_copy(zero_v, acc.at[pl.ds(sid * rpt + z * 16, 16)])
            return c
        lax.fori_loop(0, rpt // 16, zbody, 0)
        plsc.subcore_barrier()

        def ebody(j, c):
            pltpu.sync_copy(ones_v, acc.at[dst_v.at[j]], add=True)
            return c
        lax.fori_loop(0, ch, ebody, 0)
        plsc.subcore_barrier()

        pltpu.sync_copy(acc.at[pl.ds(sid * rpt, rpt)],
                        out_h.at[cid, pl.ds(sid * rpt, rpt)])

    return k(dstp)


def _sc_scatter(table, srcp, dstp, d):
    ch = srcp.shape[1]
    rpt = _NPAD // _NSUB
    zr = 16
    mesh = plsc.VectorSubcoreMesh(core_axis_name="c", subcore_axis_name="s")

    @functools.partial(
        pl.kernel,
        out_type=jax.ShapeDtypeStruct((2, _NPAD, d), jnp.float32),
        mesh=mesh,
        scratch_types=[
            pltpu.VMEM((ch, _K), jnp.int32),
            pltpu.VMEM((ch, _K), jnp.int32),
            pltpu.VMEM((_K, d), jnp.float32),
            pltpu.VMEM((zr, d), jnp.float32),
            pltpu.VMEM_SHARED((_NPAD, d), jnp.float32),
            pltpu.SemaphoreType.DMA,
        ],
        compiler_params=pltpu.CompilerParams(use_tc_tiling_on_sc=False),
    )
    def k(table_h, src_h, dst_h, out_h, src_v, dst_v, rows_v, zero_v, acc, sem):
        cid = lax.axis_index("c")
        sid = lax.axis_index("s")
        wid = cid * _NSUB + sid

        zero = jnp.zeros((16,), jnp.float32)
        for r in range(zr):
            for cc in range(d // 16):
                zero_v[r, pl.ds(cc * 16, 16)] = zero

        pltpu.sync_copy(src_h.at[wid], src_v)
        pltpu.sync_copy(dst_h.at[wid], dst_v)

        def zbody(z, c):
            pltpu.sync_copy(zero_v, acc.at[pl.ds(sid * rpt + z * zr, zr)])
            return c
        lax.fori_loop(0, rpt // zr, zbody, 0)
        plsc.subcore_barrier()

        def ebody(j, c):
            pltpu.async_copy(table_h.at[src_v.at[j]], rows_v, sem).wait()
            pltpu.sync_copy(rows_v, acc.at[dst_v.at[j]], add=True)
            return c
        lax.fori_loop(0, ch, ebody, 0)
        plsc.subcore_barrier()

        pltpu.sync_copy(acc.at[pl.ds(sid * rpt, rpt)],
                        out_h.at[cid, pl.ds(sid * rpt, rpt)])

    return k(table, srcp, dstp)


def _tc1(x, w1, degn):
    def body(x_ref, w_ref, dg_ref, h_ref, dv_ref):
        deg = dg_ref[0] + dg_ref[1] + 1.0
        dinv = lax.rsqrt(deg)
        h_ref[...] = jnp.dot(x_ref[...], w_ref[...],
                             preferred_element_type=jnp.float32) * dinv
        dv_ref[...] = dinv

    return pl.pallas_call(
        body,
        grid=(_N // _BLK,),
        in_specs=[
            pl.BlockSpec((_BLK, 128), lambda i: (i, 0)),
            pl.BlockSpec((128, 128), lambda i: (0, 0)),
            pl.BlockSpec((2, _BLK, 1), lambda i: (0, i, 0)),
        ],
        out_specs=[
            pl.BlockSpec((_BLK, 128), lambda i: (i, 0)),
            pl.BlockSpec((_BLK, 1), lambda i: (i, 0)),
        ],
        out_shape=[
            jax.ShapeDtypeStruct((_N, 128), jnp.float32),
            jax.ShapeDtypeStruct((_N, 1), jnp.float32),
        ],
    )(x, w1, degn)


def _tc2(p1, h1p, dinv, b1, w2):
    def body(p_ref, h_ref, dv_ref, b_ref, w_ref, o_ref):
        dinv = dv_ref[...]
        z = dinv * (p_ref[0] + p_ref[1] + h_ref[...]) + b_ref[...]
        h2 = jnp.maximum(z, 0.0)
        o_ref[...] = jnp.dot(h2, w_ref[...],
                             preferred_element_type=jnp.float32) * dinv

    return pl.pallas_call(
        body,
        grid=(_N // _BLK,),
        in_specs=[
            pl.BlockSpec((2, _BLK, 128), lambda i: (0, i, 0)),
            pl.BlockSpec((_BLK, 128), lambda i: (i, 0)),
            pl.BlockSpec((_BLK, 1), lambda i: (i, 0)),
            pl.BlockSpec((1, 128), lambda i: (0, 0)),
            pl.BlockSpec((128, 64), lambda i: (0, 0)),
        ],
        out_specs=pl.BlockSpec((_BLK, 64), lambda i: (i, 0)),
        out_shape=jax.ShapeDtypeStruct((_N, 64), jnp.float32),
    )(p1, h1p, dinv, b1, w2)


def _tc3(p2, g2p, dinv, b2):
    def body(p_ref, g_ref, dv_ref, b_ref, o_ref):
        z = dv_ref[...] * (p_ref[0] + p_ref[1] + g_ref[...]) + b_ref[...]
        m = jnp.max(z, axis=1, keepdims=True)
        e = jnp.exp(z - m)
        s = jnp.sum(e, axis=1, keepdims=True)
        o_ref[...] = (z - m) - jnp.log(s)

    return pl.pallas_call(
        body,
        grid=(_N // _BLK,),
        in_specs=[
            pl.BlockSpec((2, _BLK, 64), lambda i: (0, i, 0)),
            pl.BlockSpec((_BLK, 64), lambda i: (i, 0)),
            pl.BlockSpec((_BLK, 1), lambda i: (i, 0)),
            pl.BlockSpec((1, 64), lambda i: (0, 0)),
        ],
        out_specs=pl.BlockSpec((_BLK, 64), lambda i: (i, 0)),
        out_shape=jax.ShapeDtypeStruct((_N, 64), jnp.float32),
    )(p2, g2p, dinv, b2)


def kernel(x, edge_index, W1, b1, W2, b2):
    ei = edge_index.astype(jnp.int32)
    src, dst = ei[0], ei[1]
    e = src.shape[0]
    ept = -(-e // _NW)
    ch = -(-ept // _K)
    pad = _NW * ch * _K - e
    srcp = jnp.concatenate([src, jnp.zeros((pad,), jnp.int32)]).reshape(_NW, ch, _K)
    dstp = jnp.concatenate([dst, jnp.full((pad,), _N, jnp.int32)]).reshape(_NW, ch, _K)

    degp = _sc_degree(dstp)
    degn = degp[:, :_N, 0:1]
    h1p, dinv = _tc1(x, W1, degn)
    p1 = _sc_scatter(h1p, srcp, dstp, 128)[:, :_N, :]
    g2p = _tc2(p1, h1p, dinv, b1.reshape(1, 128), W2)
    p2 = _sc_scatter(g2p, srcp, dstp, 64)[:, :_N, :]
    return _tc3(p2, g2p, dinv, b2.reshape(1, 64))
